# Initial kernel scaffold; baseline (speedup 1.0000x reference)
import jax
import jax.numpy as jnp
from jax.experimental import pallas as pl


def kernel(nodes, edges, W1, b1, Wq, bq, Wk, bk, Wv, bv, Wo, bo):
    raise NotImplementedError("write your pallas kernel here")


# speedup vs baseline: 796.9686x; 796.9686x over previous
import jax
import jax.numpy as jnp
from jax.experimental import pallas as pl

_N = 768
_D = 256


def _tdot(a, b):
    return jax.lax.dot_general(
        a, b, (((0,), (0,)), ((), ())), preferred_element_type=jnp.float32)


def _topo_kernel(edges_ref, nodes_ref, w1_ref, b1_ref, wq_ref, bq_ref,
                 wk_ref, bk_ref, wv_ref, bv_ref, wo_ref, bo_ref, out_ref):
    f32 = jnp.float32
    ii = jax.lax.broadcasted_iota(jnp.int32, (_N, _N), 0)
    jj = jax.lax.broadcasted_iota(jnp.int32, (_N, _N), 1)
    eu = jnp.where(ii < jj, edges_ref[...], 0.0)

    deg = _tdot(eu, jnp.ones((_N, 1), f32)) + 1.0
    dinv = jax.lax.rsqrt(deg)

    nodes = nodes_ref[...]
    h = jnp.dot(nodes, w1_ref[...], preferred_element_type=f32)
    g = dinv * h
    agg = dinv * (_tdot(eu, g) + g) + b1_ref[...]
    x = nodes + jnp.maximum(agg, 0.0)

    q = jnp.dot(x, wq_ref[...], preferred_element_type=f32) + bq_ref[...]
    k = jnp.dot(x, wk_ref[...], preferred_element_type=f32) + bk_ref[...]
    v = jnp.dot(x, wv_ref[...], preferred_element_type=f32) + bv_ref[...]

    scale = 1.0 / jnp.sqrt(jnp.asarray(_D, f32))
    logits = jax.lax.dot_general(
        q, k, (((1,), (1,)), ((), ())), preferred_element_type=f32) * scale
    m = jnp.max(logits, axis=1, keepdims=True)
    p = jnp.exp(logits - m)
    s = jnp.sum(p, axis=1, keepdims=True)
    av = jnp.dot(p, v, preferred_element_type=f32) / s
    out_ref[...] = jnp.dot(av, wo_ref[...], preferred_element_type=f32) + bo_ref[...]


def kernel(nodes, edges, W1, b1, Wq, bq, Wk, bk, Wv, bv, Wo, bo):
    b1r, bqr, bkr, bvr, bor = (b.reshape(1, _D) for b in (b1, bq, bk, bv, bo))
    return pl.pallas_call(
        _topo_kernel,
        out_shape=jax.ShapeDtypeStruct((_N, _D), jnp.float32),
    )(edges, nodes, W1, b1r, Wq, bqr, Wk, bkr, Wv, bvr, Wo, bor)
